# baseline (speedup 1.0000x reference)
import jax
import jax.numpy as jnp
from jax.experimental import pallas as pl
from jax.experimental.pallas import tpu as pltpu

_EPS = 1e-6
_LANES = 3840
_GRID = 28
_CHUNK = 8


def _rotl(x, k):
    return jnp.concatenate([x[..., k:], x[..., :k]], axis=-1)


def _rotr(x, k):
    return jnp.concatenate([x[..., -k:], x[..., :-k]], axis=-1)


def _loss_kernel(p_ref, t_ref, o_ref):
    bn = p_ref.shape[0]
    f1 = jnp.float32(1.0)
    f0 = jnp.float32(0.0)

    c = jax.lax.broadcasted_iota(jnp.int32, (1, _LANES), 1) % 30
    rep_lane = (c >= 5) & (c < 10)
    wh_lane = (c == 2) | (c == 3) | (c == 7) | (c == 8)
    c0_lane = c == 0
    c4_lane = c == 4
    box0 = c < 5
    box1 = rep_lane
    cls_lane = c >= 10
    coord0 = (c < 4) & ~(c == 4)
    a_const = jnp.where(c < 4, 5.0, jnp.where(c4_lane, 1.0,
                        jnp.where(cls_lane, 1.0, 0.0))).astype(jnp.float32)
    b_const = jnp.where(c < 4, -5.0, jnp.where(c4_lane, -1.0,
                        jnp.where((c >= 5) & (c < 9), 5.0,
                        jnp.where(c == 9, 1.0, 0.0)))).astype(jnp.float32)
    cc_const = jnp.where(c4_lane | (c == 9), 0.5, 0.0).astype(jnp.float32)

    def chunk(j, acc):
        sl = pl.ds(j * _CHUNK, _CHUNK)
        p = p_ref[sl]
        t = t_ref[sl]

        t_rep = jnp.where(rep_lane, _rotr(t, 5), t)

        pw = _rotl(0.5 * p, 2)
        tw = _rotl(0.5 * t_rep, 2)
        ov = jnp.maximum(
            jnp.minimum(p + pw, t_rep + tw) -
            jnp.maximum(p - pw, t_rep - tw), 0.0)
        inter = ov * _rotl(ov, 1)
        area = p * _rotl(p, 1) + t_rep * _rotl(t_rep, 1)
        union = _rotl(area, 2) - inter
        iou = inter / (union + _EPS)
        m = jnp.where(iou > 0, iou, f0)
        sel0 = _rotl(m, 5) > m

        s = jnp.where(c0_lane & sel0, f1, f0)
        s = jnp.maximum(jnp.maximum(s, _rotr(s, 1)),
                        jnp.maximum(_rotr(s, 2),
                                    jnp.maximum(_rotr(s, 3), _rotr(s, 4))))
        s = jnp.maximum(s, _rotr(s, 5))

        o = jnp.where(c4_lane & (t > 0), f1, f0)
        o = jnp.maximum(jnp.maximum(o, _rotr(o, 1)),
                        jnp.maximum(_rotr(o, 2),
                                    jnp.maximum(_rotr(o, 3), _rotr(o, 4))))
        o = jnp.maximum(jnp.maximum(o, _rotl(o, 4)),
                        jnp.maximum(_rotr(o, 5), _rotr(o, 10)))
        o = jnp.maximum(o, _rotr(o, 11))

        u = jnp.where(wh_lane, jnp.sqrt(jnp.maximum(p, _EPS)), p)
        v = jnp.where(wh_lane, jnp.sqrt(jnp.maximum(t_rep, _EPS)), t_rep)
        d = u - v
        base = d * d

        w_all = o * (a_const + b_const * s)
        conf2 = cc_const * (p * p)
        return acc + (w_all * (base - conf2) + conf2)

    def body(j, acc):
        acc = chunk(2 * j, acc)
        return chunk(2 * j + 1, acc)

    acc = jax.lax.fori_loop(
        0, bn // (2 * _CHUNK), body,
        jnp.zeros((_CHUNK, _LANES), jnp.float32))
    o_ref[...] = jnp.sum(acc, axis=(0, 1), keepdims=True).reshape(1, 1, 1)


def kernel(predictions, targets):
    n = predictions.shape[0]
    p2 = predictions.reshape(-1, _LANES)
    t2 = targets.reshape(-1, _LANES)
    rows = p2.shape[0]
    br = rows // _GRID
    partials = pl.pallas_call(
        _loss_kernel,
        grid=(_GRID,),
        in_specs=[
            pl.BlockSpec((br, _LANES), lambda i: (i, 0)),
            pl.BlockSpec((br, _LANES), lambda i: (i, 0)),
        ],
        out_specs=pl.BlockSpec((1, 1, 1), lambda i: (i, 0, 0)),
        out_shape=jax.ShapeDtypeStruct((_GRID, 1, 1), jnp.float32),
        compiler_params=pltpu.CompilerParams(
            dimension_semantics=("parallel",)),
    )(p2, t2)
    return jnp.sum(partials) / n


# speedup vs baseline: 1.1513x; 1.1513x over previous
import jax
import jax.numpy as jnp
from jax.experimental import pallas as pl
from jax.experimental.pallas import tpu as pltpu

_EPS = 1e-6
_LANES = 3840
_GRID = 28
_CHUNK = 16


def _rotl(x, k):
    return jnp.concatenate([x[..., k:], x[..., :k]], axis=-1)


def _rotr(x, k):
    return jnp.concatenate([x[..., -k:], x[..., :-k]], axis=-1)


def _loss_kernel(p_ref, t_ref, o_ref):
    bn = p_ref.shape[0]
    f1 = jnp.float32(1.0)
    f0 = jnp.float32(0.0)

    c = jax.lax.broadcasted_iota(jnp.int32, (1, _LANES), 1) % 30
    rep_lane = (c >= 5) & (c < 10)
    wh_lane = (c == 2) | (c == 3) | (c == 7) | (c == 8)
    c0_lane = c == 0
    c4_lane = c == 4
    box0 = c < 5
    box1 = rep_lane
    cls_lane = c >= 10
    coord0 = (c < 4) & ~(c == 4)
    a_const = jnp.where(c < 4, 5.0, jnp.where(c4_lane, 1.0,
                        jnp.where(cls_lane, 1.0, 0.0))).astype(jnp.float32)
    b_const = jnp.where(c < 4, -5.0, jnp.where(c4_lane, -1.0,
                        jnp.where((c >= 5) & (c < 9), 5.0,
                        jnp.where(c == 9, 1.0, 0.0)))).astype(jnp.float32)
    cc_const = jnp.where(c4_lane | (c == 9), 0.5, 0.0).astype(jnp.float32)

    def chunk(j, acc):
        sl = pl.ds(j * _CHUNK, _CHUNK)
        p = p_ref[sl]
        t = t_ref[sl]

        t_rep = jnp.where(rep_lane, _rotr(t, 5), t)

        pw = _rotl(0.5 * p, 2)
        tw = _rotl(0.5 * t_rep, 2)
        ov = jnp.maximum(
            jnp.minimum(p + pw, t_rep + tw) -
            jnp.maximum(p - pw, t_rep - tw), 0.0)
        inter = ov * _rotl(ov, 1)
        area = p * _rotl(p, 1) + t_rep * _rotl(t_rep, 1)
        union = _rotl(area, 2) - inter
        iou = inter / (union + _EPS)
        m = jnp.where(iou > 0, iou, f0)
        sel0 = _rotl(m, 5) > m

        s = jnp.where(c0_lane & sel0, f1, f0)
        s = jnp.maximum(jnp.maximum(s, _rotr(s, 1)),
                        jnp.maximum(_rotr(s, 2),
                                    jnp.maximum(_rotr(s, 3), _rotr(s, 4))))
        s = jnp.maximum(s, _rotr(s, 5))

        o = jnp.where(c4_lane & (t > 0), f1, f0)
        o = jnp.maximum(jnp.maximum(o, _rotr(o, 1)),
                        jnp.maximum(_rotr(o, 2),
                                    jnp.maximum(_rotr(o, 3), _rotr(o, 4))))
        o = jnp.maximum(jnp.maximum(o, _rotl(o, 4)),
                        jnp.maximum(_rotr(o, 5), _rotr(o, 10)))
        o = jnp.maximum(o, _rotr(o, 11))

        u = jnp.where(wh_lane, jnp.sqrt(jnp.maximum(p, _EPS)), p)
        v = jnp.where(wh_lane, jnp.sqrt(jnp.maximum(t_rep, _EPS)), t_rep)
        d = u - v
        base = d * d

        w_all = o * (a_const + b_const * s)
        conf2 = cc_const * (p * p)
        return acc + (w_all * (base - conf2) + conf2)

    def body(j, acc):
        acc = chunk(2 * j, acc)
        return chunk(2 * j + 1, acc)

    acc = jax.lax.fori_loop(
        0, bn // (2 * _CHUNK), body,
        jnp.zeros((_CHUNK, _LANES), jnp.float32))
    o_ref[...] = jnp.sum(acc, axis=(0, 1), keepdims=True).reshape(1, 1, 1)


def kernel(predictions, targets):
    n = predictions.shape[0]
    p2 = predictions.reshape(-1, _LANES)
    t2 = targets.reshape(-1, _LANES)
    rows = p2.shape[0]
    br = rows // _GRID
    partials = pl.pallas_call(
        _loss_kernel,
        grid=(_GRID,),
        in_specs=[
            pl.BlockSpec((br, _LANES), lambda i: (i, 0)),
            pl.BlockSpec((br, _LANES), lambda i: (i, 0)),
        ],
        out_specs=pl.BlockSpec((1, 1, 1), lambda i: (i, 0, 0)),
        out_shape=jax.ShapeDtypeStruct((_GRID, 1, 1), jnp.float32),
        compiler_params=pltpu.CompilerParams(
            dimension_semantics=("parallel",)),
    )(p2, t2)
    return jnp.sum(partials) / n


# speedup vs baseline: 1.1820x; 1.0267x over previous
import jax
import jax.numpy as jnp
from jax.experimental import pallas as pl
from jax.experimental.pallas import tpu as pltpu

_EPS = 1e-6
_LANES = 3840
_GRID = 28
_CHUNK = 112


def _rotl(x, k):
    return jnp.concatenate([x[..., k:], x[..., :k]], axis=-1)


def _rotr(x, k):
    return jnp.concatenate([x[..., -k:], x[..., :-k]], axis=-1)


def _loss_kernel(p_ref, t_ref, o_ref):
    bn = p_ref.shape[0]
    f1 = jnp.float32(1.0)
    f0 = jnp.float32(0.0)

    c = jax.lax.broadcasted_iota(jnp.int32, (1, _LANES), 1) % 30
    rep_lane = (c >= 5) & (c < 10)
    wh_lane = (c == 2) | (c == 3) | (c == 7) | (c == 8)
    c0_lane = c == 0
    c4_lane = c == 4
    box0 = c < 5
    box1 = rep_lane
    cls_lane = c >= 10
    coord0 = (c < 4) & ~(c == 4)
    a_const = jnp.where(c < 4, 5.0, jnp.where(c4_lane, 1.0,
                        jnp.where(cls_lane, 1.0, 0.0))).astype(jnp.float32)
    b_const = jnp.where(c < 4, -5.0, jnp.where(c4_lane, -1.0,
                        jnp.where((c >= 5) & (c < 9), 5.0,
                        jnp.where(c == 9, 1.0, 0.0)))).astype(jnp.float32)
    cc_const = jnp.where(c4_lane | (c == 9), 0.5, 0.0).astype(jnp.float32)

    def chunk(j, acc):
        sl = pl.ds(j * _CHUNK, _CHUNK)
        p = p_ref[sl]
        t = t_ref[sl]

        t_rep = jnp.where(rep_lane, _rotr(t, 5), t)

        pw = _rotl(0.5 * p, 2)
        tw = _rotl(0.5 * t_rep, 2)
        ov = jnp.maximum(
            jnp.minimum(p + pw, t_rep + tw) -
            jnp.maximum(p - pw, t_rep - tw), 0.0)
        inter = ov * _rotl(ov, 1)
        area = p * _rotl(p, 1) + t_rep * _rotl(t_rep, 1)
        union = _rotl(area, 2) - inter
        iou = inter / (union + _EPS)
        m = jnp.where(iou > 0, iou, f0)
        sel0 = _rotl(m, 5) > m

        s = jnp.where(c0_lane & sel0, f1, f0)
        s = jnp.maximum(jnp.maximum(s, _rotr(s, 1)),
                        jnp.maximum(_rotr(s, 2),
                                    jnp.maximum(_rotr(s, 3), _rotr(s, 4))))
        s = jnp.maximum(s, _rotr(s, 5))

        o = jnp.where(c4_lane & (t > 0), f1, f0)
        o = jnp.maximum(jnp.maximum(o, _rotr(o, 1)),
                        jnp.maximum(_rotr(o, 2),
                                    jnp.maximum(_rotr(o, 3), _rotr(o, 4))))
        o = jnp.maximum(jnp.maximum(o, _rotl(o, 4)),
                        jnp.maximum(_rotr(o, 5), _rotr(o, 10)))
        o = jnp.maximum(o, _rotr(o, 11))

        u = jnp.where(wh_lane, jnp.sqrt(jnp.maximum(p, _EPS)), p)
        v = jnp.where(wh_lane, jnp.sqrt(jnp.maximum(t_rep, _EPS)), t_rep)
        d = u - v
        base = d * d

        w_all = o * (a_const + b_const * s)
        conf2 = cc_const * (p * p)
        return acc + (w_all * (base - conf2) + conf2)

    def body(j, acc):
        acc = chunk(2 * j, acc)
        return chunk(2 * j + 1, acc)

    acc = jax.lax.fori_loop(
        0, bn // (2 * _CHUNK), body,
        jnp.zeros((_CHUNK, _LANES), jnp.float32))
    o_ref[...] = jnp.sum(acc, axis=(0, 1), keepdims=True).reshape(1, 1, 1)


def kernel(predictions, targets):
    n = predictions.shape[0]
    p2 = predictions.reshape(-1, _LANES)
    t2 = targets.reshape(-1, _LANES)
    rows = p2.shape[0]
    br = rows // _GRID
    partials = pl.pallas_call(
        _loss_kernel,
        grid=(_GRID,),
        in_specs=[
            pl.BlockSpec((br, _LANES), lambda i: (i, 0)),
            pl.BlockSpec((br, _LANES), lambda i: (i, 0)),
        ],
        out_specs=pl.BlockSpec((1, 1, 1), lambda i: (i, 0, 0)),
        out_shape=jax.ShapeDtypeStruct((_GRID, 1, 1), jnp.float32),
        compiler_params=pltpu.CompilerParams(
            dimension_semantics=("parallel",)),
    )(p2, t2)
    return jnp.sum(partials) / n


# speedup vs baseline: 1.2162x; 1.0289x over previous
import jax
import jax.numpy as jnp
from jax.experimental import pallas as pl
from jax.experimental.pallas import tpu as pltpu

_EPS = 1e-6
_LANES = 3840
_GRID = 28
_CHUNK = 112


def _rotl(x, k):
    return jnp.concatenate([x[..., k:], x[..., :k]], axis=-1)


def _rotr(x, k):
    return jnp.concatenate([x[..., -k:], x[..., :-k]], axis=-1)


def _loss_kernel(p_ref, t_ref, o_ref):
    bn = p_ref.shape[0]
    f1 = jnp.float32(1.0)
    f0 = jnp.float32(0.0)

    c = jax.lax.broadcasted_iota(jnp.int32, (1, _LANES), 1) % 30
    rep_lane = (c >= 5) & (c < 10)
    wh_lane = (c == 2) | (c == 3) | (c == 7) | (c == 8)
    c0_lane = c == 0
    c4_lane = c == 4
    box0 = c < 5
    box1 = rep_lane
    cls_lane = c >= 10
    coord0 = (c < 4) & ~(c == 4)
    a_const = jnp.where(c < 4, 5.0, jnp.where(c4_lane, 1.0,
                        jnp.where(cls_lane, 1.0, 0.0))).astype(jnp.float32)
    b_const = jnp.where(c < 4, -5.0, jnp.where(c4_lane, -1.0,
                        jnp.where((c >= 5) & (c < 9), 5.0,
                        jnp.where(c == 9, 1.0, 0.0)))).astype(jnp.float32)
    cc_const = jnp.where(c4_lane | (c == 9), 0.5, 0.0).astype(jnp.float32)

    def chunk():
        p = p_ref[...]
        t = t_ref[...]

        t_rep = jnp.where(rep_lane, _rotr(t, 5), t)

        pw = _rotl(0.5 * p, 2)
        tw = _rotl(0.5 * t_rep, 2)
        ov = jnp.maximum(
            jnp.minimum(p + pw, t_rep + tw) -
            jnp.maximum(p - pw, t_rep - tw), 0.0)
        inter = ov * _rotl(ov, 1)
        area = p * _rotl(p, 1) + t_rep * _rotl(t_rep, 1)
        union = _rotl(area, 2) - inter
        iou = inter / (union + _EPS)
        m = jnp.where(iou > 0, iou, f0)
        sel0 = _rotl(m, 5) > m

        s = jnp.where(c0_lane & sel0, f1, f0)
        s = jnp.maximum(jnp.maximum(s, _rotr(s, 1)),
                        jnp.maximum(_rotr(s, 2),
                                    jnp.maximum(_rotr(s, 3), _rotr(s, 4))))
        s = jnp.maximum(s, _rotr(s, 5))

        o = jnp.where(c4_lane & (t > 0), f1, f0)
        o = jnp.maximum(jnp.maximum(o, _rotr(o, 1)),
                        jnp.maximum(_rotr(o, 2),
                                    jnp.maximum(_rotr(o, 3), _rotr(o, 4))))
        o = jnp.maximum(jnp.maximum(o, _rotl(o, 4)),
                        jnp.maximum(_rotr(o, 5), _rotr(o, 10)))
        o = jnp.maximum(o, _rotr(o, 11))

        u = jnp.where(wh_lane, jnp.sqrt(jnp.maximum(p, _EPS)), p)
        v = jnp.where(wh_lane, jnp.sqrt(jnp.maximum(t_rep, _EPS)), t_rep)
        d = u - v
        base = d * d

        w_all = o * (a_const + b_const * s)
        conf2 = cc_const * (p * p)
        return w_all * (base - conf2) + conf2

    acc = chunk()
    o_ref[...] = jnp.sum(acc, axis=(0, 1), keepdims=True).reshape(1, 1, 1)


def kernel(predictions, targets):
    n = predictions.shape[0]
    p2 = predictions.reshape(-1, _LANES)
    t2 = targets.reshape(-1, _LANES)
    rows = p2.shape[0]
    br = rows // _GRID
    partials = pl.pallas_call(
        _loss_kernel,
        grid=(_GRID,),
        in_specs=[
            pl.BlockSpec((br, _LANES), lambda i: (i, 0)),
            pl.BlockSpec((br, _LANES), lambda i: (i, 0)),
        ],
        out_specs=pl.BlockSpec((1, 1, 1), lambda i: (i, 0, 0)),
        out_shape=jax.ShapeDtypeStruct((_GRID, 1, 1), jnp.float32),
        compiler_params=pltpu.CompilerParams(
            dimension_semantics=("parallel",)),
    )(p2, t2)
    return jnp.sum(partials) / n


# speedup vs baseline: 1.2334x; 1.0141x over previous
import jax
import jax.numpy as jnp
from jax.experimental import pallas as pl
from jax.experimental.pallas import tpu as pltpu

_EPS = 1e-6
_LANES = 3840
_GRID = 28
_CHUNK = 112


def _rotl(x, k):
    z = jnp.zeros(x.shape[:-1] + (k,), x.dtype)
    return jnp.concatenate([x[..., k:], z], axis=-1)


def _rotr(x, k):
    z = jnp.zeros(x.shape[:-1] + (k,), x.dtype)
    return jnp.concatenate([z, x[..., :-k]], axis=-1)


def _loss_kernel(p_ref, t_ref, o_ref):
    bn = p_ref.shape[0]
    f1 = jnp.float32(1.0)
    f0 = jnp.float32(0.0)

    c = jax.lax.broadcasted_iota(jnp.int32, (1, _LANES), 1) % 30
    rep_lane = (c >= 5) & (c < 10)
    wh_lane = (c == 2) | (c == 3) | (c == 7) | (c == 8)
    c0_lane = c == 0
    c4_lane = c == 4
    box0 = c < 5
    box1 = rep_lane
    cls_lane = c >= 10
    coord0 = (c < 4) & ~(c == 4)
    a_const = jnp.where(c < 4, 5.0, jnp.where(c4_lane, 1.0,
                        jnp.where(cls_lane, 1.0, 0.0))).astype(jnp.float32)
    b_const = jnp.where(c < 4, -5.0, jnp.where(c4_lane, -1.0,
                        jnp.where((c >= 5) & (c < 9), 5.0,
                        jnp.where(c == 9, 1.0, 0.0)))).astype(jnp.float32)
    cc_const = jnp.where(c4_lane | (c == 9), 0.5, 0.0).astype(jnp.float32)

    def chunk():
        p = p_ref[...]
        t = t_ref[...]

        t_rep = jnp.where(rep_lane, _rotr(t, 5), t)

        pw = _rotl(0.5 * p, 2)
        tw = _rotl(0.5 * t_rep, 2)
        ov = jnp.maximum(
            jnp.minimum(p + pw, t_rep + tw) -
            jnp.maximum(p - pw, t_rep - tw), 0.0)
        inter = ov * _rotl(ov, 1)
        area = p * _rotl(p, 1) + t_rep * _rotl(t_rep, 1)
        union = _rotl(area, 2) - inter
        iou = inter / (union + _EPS)
        m = jnp.where(iou > 0, iou, f0)
        sel0 = _rotl(m, 5) > m

        s = jnp.where(c0_lane & sel0, f1, f0)
        s = jnp.maximum(jnp.maximum(s, _rotr(s, 1)),
                        jnp.maximum(_rotr(s, 2),
                                    jnp.maximum(_rotr(s, 3), _rotr(s, 4))))
        s = jnp.maximum(s, _rotr(s, 5))

        o = jnp.where(c4_lane & (t > 0), f1, f0)
        o = jnp.maximum(jnp.maximum(o, _rotr(o, 1)),
                        jnp.maximum(_rotr(o, 2),
                                    jnp.maximum(_rotr(o, 3), _rotr(o, 4))))
        o = jnp.maximum(jnp.maximum(o, _rotl(o, 4)),
                        jnp.maximum(_rotr(o, 5), _rotr(o, 10)))
        o = jnp.maximum(o, _rotr(o, 11))

        u = jnp.where(wh_lane, jnp.sqrt(jnp.maximum(p, _EPS)), p)
        v = jnp.where(wh_lane, jnp.sqrt(jnp.maximum(t_rep, _EPS)), t_rep)
        d = u - v
        base = d * d

        w_all = o * (a_const + b_const * s)
        conf2 = cc_const * (p * p)
        return w_all * (base - conf2) + conf2

    acc = chunk()
    o_ref[...] = jnp.sum(acc, axis=(0, 1), keepdims=True).reshape(1, 1, 1)


def kernel(predictions, targets):
    n = predictions.shape[0]
    p2 = predictions.reshape(-1, _LANES)
    t2 = targets.reshape(-1, _LANES)
    rows = p2.shape[0]
    br = rows // _GRID
    partials = pl.pallas_call(
        _loss_kernel,
        grid=(_GRID,),
        in_specs=[
            pl.BlockSpec((br, _LANES), lambda i: (i, 0)),
            pl.BlockSpec((br, _LANES), lambda i: (i, 0)),
        ],
        out_specs=pl.BlockSpec((1, 1, 1), lambda i: (i, 0, 0)),
        out_shape=jax.ShapeDtypeStruct((_GRID, 1, 1), jnp.float32),
        compiler_params=pltpu.CompilerParams(
            dimension_semantics=("parallel",)),
    )(p2, t2)
    return jnp.sum(partials) / n


# speedup vs baseline: 1.2874x; 1.0438x over previous
import jax
import jax.numpy as jnp
from jax.experimental import pallas as pl
from jax.experimental.pallas import tpu as pltpu

_EPS = 1e-6
_GROUP = 30
_LANES = 3840
_GRID = 28


def _shl(x, k):
    z = jnp.zeros((x.shape[0], k), x.dtype)
    return jnp.concatenate([x[:, k:], z], axis=1)


def _shr(x, k):
    z = jnp.zeros((x.shape[0], k), x.dtype)
    return jnp.concatenate([z, x[:, :-k]], axis=1)


def _block_loss(p, t):
    lane = jax.lax.broadcasted_iota(jnp.int32, (1, _LANES), 1)
    c = lane % _GROUP
    box_lane = c < 10
    wh_lane = (c == 2) | (c == 3) | (c == 7) | (c == 8)
    conf_lane = (c == 4) | (c == 9)
    xy_lane = (c == 0) | (c == 1) | (c == 5) | (c == 6)
    coef = jnp.where(wh_lane | xy_lane, 5.0, 1.0).astype(jnp.float32)

    t_rep = jnp.where((c >= 5) & box_lane, _shr(t, 5), t)

    pw = _shl(0.5 * p, 2)
    tw = _shl(0.5 * t_rep, 2)
    ov = jnp.maximum(
        jnp.minimum(p + pw, t_rep + tw) - jnp.maximum(p - pw, t_rep - tw),
        0.0)
    inter = ov * _shl(ov, 1)
    area = p * _shl(p, 1) + t_rep * _shl(t_rep, 1)
    union = _shl(area, 2) - inter
    iou = inter / (union + _EPS)
    m = jnp.where(iou > 0, iou, 0.0)
    sel0 = _shl(m, 5) > m

    s = jnp.where((c == 0) & sel0, 1.0, 0.0)
    s = jnp.maximum(s, _shr(s, 1))
    s = jnp.maximum(s, _shr(s, 2))
    s = jnp.maximum(s, _shr(s, 4))
    sel = jnp.maximum(s, _shr(s, 2))
    selw = jnp.where(c < 5, 1.0 - sel, sel)

    o = jnp.where((c == 4) & (t > 0), 1.0, 0.0)
    o = jnp.maximum(o, _shr(o, 1))
    o = jnp.maximum(o, _shr(o, 2))
    o = jnp.maximum(o, _shr(o, 4))
    o = jnp.maximum(o, _shr(o, 8))
    o = jnp.maximum(o, _shr(o, 10))
    obj = jnp.maximum(o, _shl(o, 4))

    a = p - t_rep
    a = a * a
    w_ = jnp.sqrt(jnp.maximum(p, _EPS)) - jnp.sqrt(jnp.maximum(t_rep, _EPS))
    w_ = w_ * w_
    base = jnp.where(wh_lane, w_, a)

    wsel = jnp.where(box_lane, selw, 1.0)
    contrib = base * (obj * wsel * coef)

    psq = p * p
    noobj = 0.5 * psq * (1.0 - obj * selw)
    contrib = contrib + jnp.where(conf_lane, noobj, 0.0)
    return jnp.sum(contrib, axis=(0, 1), keepdims=True)


def _loss_kernel(p_ref, t_ref, o_ref):
    o_ref[...] = _block_loss(p_ref[...], t_ref[...]).reshape(1, 1, 1)


def kernel(predictions, targets):
    n = predictions.shape[0]
    p2 = predictions.reshape(-1, _LANES)
    t2 = targets.reshape(-1, _LANES)
    rows = p2.shape[0]
    br = rows // _GRID
    partials = pl.pallas_call(
        _loss_kernel,
        grid=(_GRID,),
        in_specs=[
            pl.BlockSpec((br, _LANES), lambda i: (i, 0)),
            pl.BlockSpec((br, _LANES), lambda i: (i, 0)),
        ],
        out_specs=pl.BlockSpec((1, 1, 1), lambda i: (i, 0, 0)),
        out_shape=jax.ShapeDtypeStruct((_GRID, 1, 1), jnp.float32),
        compiler_params=pltpu.CompilerParams(
            dimension_semantics=("parallel",)),
    )(p2, t2)
    return jnp.sum(partials) / n


# speedup vs baseline: 1.2875x; 1.0001x over previous
import jax
import jax.numpy as jnp
from jax.experimental import pallas as pl
from jax.experimental.pallas import tpu as pltpu

_EPS = 1e-6
_GROUP = 30
_LANES = 3840
_GRID = 56


def _shl(x, k):
    z = jnp.zeros((x.shape[0], k), x.dtype)
    return jnp.concatenate([x[:, k:], z], axis=1)


def _shr(x, k):
    z = jnp.zeros((x.shape[0], k), x.dtype)
    return jnp.concatenate([z, x[:, :-k]], axis=1)


def _block_loss(p, t):
    lane = jax.lax.broadcasted_iota(jnp.int32, (1, _LANES), 1)
    c = lane % _GROUP
    box_lane = c < 10
    wh_lane = (c == 2) | (c == 3) | (c == 7) | (c == 8)
    conf_lane = (c == 4) | (c == 9)
    xy_lane = (c == 0) | (c == 1) | (c == 5) | (c == 6)
    coef = jnp.where(wh_lane | xy_lane, 5.0, 1.0).astype(jnp.float32)

    t_rep = jnp.where((c >= 5) & box_lane, _shr(t, 5), t)

    pw = _shl(0.5 * p, 2)
    tw = _shl(0.5 * t_rep, 2)
    ov = jnp.maximum(
        jnp.minimum(p + pw, t_rep + tw) - jnp.maximum(p - pw, t_rep - tw),
        0.0)
    inter = ov * _shl(ov, 1)
    area = p * _shl(p, 1) + t_rep * _shl(t_rep, 1)
    union = _shl(area, 2) - inter
    iou = inter / (union + _EPS)
    m = jnp.where(iou > 0, iou, 0.0)
    sel0 = _shl(m, 5) > m

    s = jnp.where((c == 0) & sel0, 1.0, 0.0)
    s = jnp.maximum(s, _shr(s, 1))
    s = jnp.maximum(s, _shr(s, 2))
    s = jnp.maximum(s, _shr(s, 4))
    sel = jnp.maximum(s, _shr(s, 2))
    selw = jnp.where(c < 5, 1.0 - sel, sel)

    o = jnp.where((c == 4) & (t > 0), 1.0, 0.0)
    o = jnp.maximum(o, _shr(o, 1))
    o = jnp.maximum(o, _shr(o, 2))
    o = jnp.maximum(o, _shr(o, 4))
    o = jnp.maximum(o, _shr(o, 8))
    o = jnp.maximum(o, _shr(o, 10))
    obj = jnp.maximum(o, _shl(o, 4))

    a = p - t_rep
    a = a * a
    w_ = jnp.sqrt(jnp.maximum(p, _EPS)) - jnp.sqrt(jnp.maximum(t_rep, _EPS))
    w_ = w_ * w_
    base = jnp.where(wh_lane, w_, a)

    wsel = jnp.where(box_lane, selw, 1.0)
    contrib = base * (obj * wsel * coef)

    psq = p * p
    noobj = 0.5 * psq * (1.0 - obj * selw)
    contrib = contrib + jnp.where(conf_lane, noobj, 0.0)
    return jnp.sum(contrib, axis=(0, 1), keepdims=True)


def _loss_kernel(p_ref, t_ref, o_ref):
    o_ref[...] = _block_loss(p_ref[...], t_ref[...]).reshape(1, 1, 1)


def kernel(predictions, targets):
    n = predictions.shape[0]
    p2 = predictions.reshape(-1, _LANES)
    t2 = targets.reshape(-1, _LANES)
    rows = p2.shape[0]
    br = rows // _GRID
    partials = pl.pallas_call(
        _loss_kernel,
        grid=(_GRID,),
        in_specs=[
            pl.BlockSpec((br, _LANES), lambda i: (i, 0)),
            pl.BlockSpec((br, _LANES), lambda i: (i, 0)),
        ],
        out_specs=pl.BlockSpec((1, 1, 1), lambda i: (i, 0, 0)),
        out_shape=jax.ShapeDtypeStruct((_GRID, 1, 1), jnp.float32),
        compiler_params=pltpu.CompilerParams(
            dimension_semantics=("parallel",)),
    )(p2, t2)
    return jnp.sum(partials) / n


# speedup vs baseline: 1.2888x; 1.0010x over previous
import jax
import jax.numpy as jnp
from jax.experimental import pallas as pl
from jax.experimental.pallas import tpu as pltpu

_EPS = 1e-6
_GROUP = 30
_LANES = 3840
_GRID = 28


def _shl(x, k):
    z = jnp.zeros((x.shape[0], k), x.dtype)
    return jnp.concatenate([x[:, k:], z], axis=1)


def _shr(x, k):
    z = jnp.zeros((x.shape[0], k), x.dtype)
    return jnp.concatenate([z, x[:, :-k]], axis=1)


def _block_loss(p, t):
    lane = jax.lax.broadcasted_iota(jnp.int32, (1, _LANES), 1)
    c = lane % _GROUP
    box_lane = c < 10
    wh_lane = (c == 2) | (c == 3) | (c == 7) | (c == 8)
    conf_lane = (c == 4) | (c == 9)
    xy_lane = (c == 0) | (c == 1) | (c == 5) | (c == 6)
    coef = jnp.where(wh_lane | xy_lane, 5.0, 1.0).astype(jnp.float32)

    t_rep = jnp.where((c >= 5) & box_lane, _shr(t, 5), t)

    pw = _shl(0.5 * p, 2)
    tw = _shl(0.5 * t_rep, 2)
    ov = jnp.maximum(
        jnp.minimum(p + pw, t_rep + tw) - jnp.maximum(p - pw, t_rep - tw),
        0.0)
    inter = ov * _shl(ov, 1)
    area = p * _shl(p, 1) + t_rep * _shl(t_rep, 1)
    union = _shl(area, 2) - inter
    iou = inter / (union + _EPS)
    m = jnp.where(iou > 0, iou, 0.0)
    sel0 = _shl(m, 5) > m

    s = jnp.where((c == 0) & sel0, 1.0, 0.0)
    s = jnp.maximum(s, _shr(s, 1))
    s = jnp.maximum(s, _shr(s, 2))
    s = jnp.maximum(s, _shr(s, 4))
    sel = jnp.maximum(s, _shr(s, 2))
    selw = jnp.where(c < 5, 1.0 - sel, sel)

    o = jnp.where((c == 4) & (t > 0), 1.0, 0.0)
    o = jnp.maximum(o, _shr(o, 1))
    o = jnp.maximum(o, _shr(o, 2))
    o = jnp.maximum(o, _shr(o, 4))
    o = jnp.maximum(o, _shr(o, 8))
    o = jnp.maximum(o, _shr(o, 10))
    obj = jnp.maximum(o, _shl(o, 4))

    a = p - t_rep
    a = a * a
    w_ = jnp.sqrt(jnp.maximum(p, _EPS)) - jnp.sqrt(jnp.maximum(t_rep, _EPS))
    w_ = w_ * w_
    base = jnp.where(wh_lane, w_, a)

    wsel = jnp.where(box_lane, selw, 1.0)
    contrib = base * (obj * wsel * coef)

    psq = p * p
    noobj = 0.5 * psq * (1.0 - obj * selw)
    contrib = contrib + jnp.where(conf_lane, noobj, 0.0)
    return jnp.sum(contrib, axis=(0, 1), keepdims=True)


def _loss_kernel(p_ref, t_ref, o_ref):
    o_ref[...] = _block_loss(p_ref[...], t_ref[...]).reshape(1, 1, 1)


def kernel(predictions, targets):
    n = predictions.shape[0]
    p2 = predictions.reshape(-1, _LANES)
    t2 = targets.reshape(-1, _LANES)
    rows = p2.shape[0]
    br = rows // _GRID
    partials = pl.pallas_call(
        _loss_kernel,
        grid=(_GRID,),
        in_specs=[
            pl.BlockSpec((br, _LANES), lambda i: (i, 0)),
            pl.BlockSpec((br, _LANES), lambda i: (i, 0)),
        ],
        out_specs=pl.BlockSpec((1, 1, 1), lambda i: (i, 0, 0)),
        out_shape=jax.ShapeDtypeStruct((_GRID, 1, 1), jnp.float32),
        compiler_params=pltpu.CompilerParams(
            dimension_semantics=("parallel",)),
    )(p2, t2)
    return jnp.sum(partials) / n
